# baseline (speedup 1.0000x reference)
import functools

import jax
import jax.numpy as jnp
from jax import lax
from jax.experimental import pallas as pl
from jax.experimental.pallas import tpu as pltpu
from jax.experimental.pallas import tpu_sc as plsc

NC, NS = 2, 16
NW = NC * NS
BB = 8
LANES = 128
PAD_SEQ = 56


def kernel(x, table):
    batch, seq = x.shape
    vocab, embed_dim = table.shape
    b_per_worker = batch // NW
    n_batches = b_per_worker // BB

    mesh = plsc.VectorSubcoreMesh(core_axis_name="c", subcore_axis_name="s")

    @functools.partial(
        pl.kernel,
        mesh=mesh,
        out_type=jax.ShapeDtypeStruct((batch, PAD_SEQ, LANES), table.dtype),
        scratch_types=[
            pltpu.VMEM((b_per_worker, seq), jnp.int32),
            pltpu.VMEM((2, BB, PAD_SEQ, LANES), jnp.float32),
            pltpu.SemaphoreType.DMA,
            pltpu.SemaphoreType.DMA,
            pltpu.SemaphoreType.DMA,
            pltpu.SemaphoreType.DMA,
        ],
        compiler_params=pltpu.CompilerParams(use_tc_tiling_on_sc=False),
    )
    def sc_gather(table_hbm, x_hbm, out_hbm, idx_v, rows_v, g0, g1, o0, o1):
        wid = lax.axis_index("s") * NC + lax.axis_index("c")
        b0 = wid * b_per_worker
        gsem = (g0, g1)
        osem = (o0, o1)
        pltpu.sync_copy(x_hbm.at[pl.ds(b0, b_per_worker)], idx_v)

        def gather_cp(buf, t, j):
            return pltpu.make_async_copy(
                table_hbm.at[idx_v.at[t * BB + j]],
                rows_v.at[buf].at[j].at[pl.ds(0, seq)],
                gsem[buf],
            )

        def fire(buf, t):
            @pl.loop(0, BB)
            def _(j):
                gather_cp(buf, t, j).start()

        def drain(buf, t):
            @pl.loop(0, BB)
            def _(j):
                gather_cp(buf, t, j).wait()

        def out_cp(buf, t):
            return pltpu.make_async_copy(
                rows_v.at[buf],
                out_hbm.at[pl.ds(b0 + t * BB, BB)],
                osem[buf],
            )

        fire(0, 0)
        fire(1, 1)

        @pl.loop(0, n_batches // 2 - 1)
        def _(h):
            t0 = 2 * h
            drain(0, t0)
            out_cp(0, t0).start()
            drain(1, t0 + 1)
            out_cp(1, t0 + 1).start()
            out_cp(0, t0).wait()
            fire(0, t0 + 2)
            out_cp(1, t0 + 1).wait()
            fire(1, t0 + 3)

        tl = n_batches - 2
        drain(0, tl)
        out_cp(0, tl).start()
        drain(1, tl + 1)
        out_cp(1, tl + 1).start()
        out_cp(0, tl).wait()
        out_cp(1, tl + 1).wait()

    tab_pad = jnp.pad(table, ((0, 0), (0, LANES - embed_dim)))
    out = sc_gather(tab_pad, x)
    f32max = jnp.float32(jnp.finfo(jnp.float32).max)
    return jnp.minimum(out[:, :seq, :embed_dim], f32max)


# speedup vs baseline: 1.4913x; 1.4913x over previous
import functools

import jax
import jax.numpy as jnp
from jax import lax
from jax.experimental import pallas as pl
from jax.experimental.pallas import tpu as pltpu
from jax.experimental.pallas import tpu_sc as plsc

NC, NS = 2, 16
NW = NC * NS
BB = 4
LANES = 128
PAD_SEQ = 56
VL = 16


def kernel(x, table):
    batch, seq = x.shape
    vocab, embed_dim = table.shape
    b_per_worker = batch // NW
    n_batches = b_per_worker // BB

    mesh = plsc.VectorSubcoreMesh(core_axis_name="c", subcore_axis_name="s")

    @functools.partial(
        pl.kernel,
        mesh=mesh,
        out_type=jax.ShapeDtypeStruct((batch, PAD_SEQ, LANES), table.dtype),
        scratch_types=[
            pltpu.VMEM((b_per_worker, seq), jnp.int32),
            pltpu.VMEM((2, BB, seq, embed_dim), jnp.float32),
            pltpu.VMEM((2, BB, PAD_SEQ, LANES), jnp.float32),
            pltpu.SemaphoreType.DMA,
            pltpu.SemaphoreType.DMA,
            pltpu.SemaphoreType.DMA,
            pltpu.SemaphoreType.DMA,
        ],
        compiler_params=pltpu.CompilerParams(use_tc_tiling_on_sc=False),
    )
    def sc_gather(table_hbm, x_hbm, out_hbm, idx_v, stg_v, rows_v, g0, g1, o0, o1):
        wid = lax.axis_index("s") * NC + lax.axis_index("c")
        b0 = wid * b_per_worker
        gsem = (g0, g1)
        osem = (o0, o1)
        pltpu.sync_copy(x_hbm.at[pl.ds(b0, b_per_worker)], idx_v)

        def gather_cp(buf, t, j):
            return pltpu.make_async_copy(
                table_hbm.at[idx_v.at[t * BB + j]],
                stg_v.at[buf].at[j],
                gsem[buf],
            )

        def fire(buf, t):
            @pl.loop(0, BB)
            def _(j):
                gather_cp(buf, t, j).start()

        def drain(buf, t):
            @pl.loop(0, BB)
            def _(j):
                gather_cp(buf, t, j).wait()

        def expand(buf):
            @pl.loop(0, BB)
            def _(j):
                for r in range(seq):
                    for c in range(0, embed_dim, VL):
                        rows_v.at[buf, j, r][pl.ds(c, VL)] = stg_v.at[buf, j, r][
                            pl.ds(c, VL)
                        ]

        def out_cp(buf, t):
            return pltpu.make_async_copy(
                rows_v.at[buf],
                out_hbm.at[pl.ds(b0 + t * BB, BB)],
                osem[buf],
            )

        fire(0, 0)
        fire(1, 1)

        @pl.loop(0, n_batches // 2 - 1)
        def _(h):
            t0 = 2 * h
            drain(0, t0)
            expand(0)
            out_cp(0, t0).start()
            drain(1, t0 + 1)
            expand(1)
            out_cp(1, t0 + 1).start()
            out_cp(0, t0).wait()
            fire(0, t0 + 2)
            out_cp(1, t0 + 1).wait()
            fire(1, t0 + 3)

        tl = n_batches - 2
        drain(0, tl)
        expand(0)
        out_cp(0, tl).start()
        drain(1, tl + 1)
        expand(1)
        out_cp(1, tl + 1).start()
        out_cp(0, tl).wait()
        out_cp(1, tl + 1).wait()

    out = sc_gather(table, x)
    return out[:, :seq, :embed_dim]


# speedup vs baseline: 1.4989x; 1.0051x over previous
import functools

import jax
import jax.numpy as jnp
from jax import lax
from jax.experimental import pallas as pl
from jax.experimental.pallas import tpu as pltpu
from jax.experimental.pallas import tpu_sc as plsc

NC, NS = 2, 16
NW = NC * NS
BB = 4
LANES = 128
PAD_SEQ = 56
VL = 16


def kernel(x, table):
    batch, seq = x.shape
    vocab, embed_dim = table.shape
    b_per_worker = batch // NW
    n_batches = b_per_worker // BB

    mesh = plsc.VectorSubcoreMesh(core_axis_name="c", subcore_axis_name="s")

    @functools.partial(
        pl.kernel,
        mesh=mesh,
        out_type=jax.ShapeDtypeStruct((batch, PAD_SEQ, LANES), table.dtype),
        scratch_types=[
            pltpu.VMEM((b_per_worker, seq), jnp.int32),
            pltpu.VMEM((2, BB, seq, embed_dim), jnp.float32),
            pltpu.VMEM((2, BB, seq, LANES), jnp.float32),
            pltpu.SemaphoreType.DMA,
            pltpu.SemaphoreType.DMA,
            pltpu.SemaphoreType.DMA,
            pltpu.SemaphoreType.DMA,
        ],
        compiler_params=pltpu.CompilerParams(use_tc_tiling_on_sc=False),
    )
    def sc_gather(table_hbm, x_hbm, out_hbm, idx_v, stg_v, rows_v, g0, g1, o0, o1):
        wid = lax.axis_index("s") * NC + lax.axis_index("c")
        b0 = wid * b_per_worker
        gsem = (g0, g1)
        osem = (o0, o1)
        pltpu.sync_copy(x_hbm.at[pl.ds(b0, b_per_worker)], idx_v)

        def gather_cp(buf, t, j):
            return pltpu.make_async_copy(
                table_hbm.at[idx_v.at[t * BB + j]],
                stg_v.at[buf].at[j],
                gsem[buf],
            )

        def fire(buf, t):
            @pl.loop(0, BB)
            def _(j):
                gather_cp(buf, t, j).start()

        def drain(buf, t):
            @pl.loop(0, BB)
            def _(j):
                gather_cp(buf, t, j).wait()

        def expand(buf):
            @pl.loop(0, BB)
            def _(j):
                for r in range(seq):
                    for c in range(0, embed_dim, VL):
                        rows_v.at[buf, j, r][pl.ds(c, VL)] = stg_v.at[buf, j, r][
                            pl.ds(c, VL)
                        ]

        def out_cp(buf, t):
            return pltpu.make_async_copy(
                rows_v.at[buf],
                out_hbm.at[pl.ds(b0 + t * BB, BB), pl.ds(0, seq)],
                osem[buf],
            )

        fire(0, 0)
        fire(1, 1)

        @pl.loop(0, n_batches // 2 - 1)
        def _(h):
            t0 = 2 * h
            drain(0, t0)
            expand(0)
            out_cp(0, t0).start()
            drain(1, t0 + 1)
            expand(1)
            out_cp(1, t0 + 1).start()
            out_cp(0, t0).wait()
            fire(0, t0 + 2)
            out_cp(1, t0 + 1).wait()
            fire(1, t0 + 3)

        tl = n_batches - 2
        drain(0, tl)
        expand(0)
        out_cp(0, tl).start()
        drain(1, tl + 1)
        expand(1)
        out_cp(1, tl + 1).start()
        out_cp(0, tl).wait()
        out_cp(1, tl + 1).wait()

    out = sc_gather(table, x)
    return out[:, :seq, :embed_dim]
